# baseline (speedup 1.0000x reference)
import functools

import jax
import jax.numpy as jnp
from jax import lax
from jax.experimental import pallas as pl
from jax.experimental.pallas import tpu as pltpu

_K = 512
_NEG = float("-inf")


def _stage(x, d, lanes, run=None, flip=None, asc=False):
    m = x.shape[0]
    if d >= 8:
        k = m // (2 * d)
        xr = x.reshape(k, 2, d, lanes)
        a = xr[:, 0]
        b = xr[:, 1]
        mx = jnp.maximum(a, b)
        mn = jnp.minimum(a, b)
        if run is None:
            lo, hi = (mn, mx) if asc else (mx, mn)
        else:
            run_shift = run.bit_length() - 1
            shift = run_shift - (2 * d).bit_length() + 1
            blk = lax.broadcasted_iota(jnp.int32, (k, 1, 1), 0)
            desc = ((blk >> shift) & 1) == 0
            if flip is not None:
                desc = desc != flip
            lo = jnp.where(desc, mx, mn)
            hi = jnp.where(desc, mn, mx)
        return jnp.concatenate([lo[:, None], hi[:, None]], axis=1).reshape(
            m, lanes
        )
    i = lax.broadcasted_iota(jnp.int32, (m, 1), 0)
    is_lower = (i & d) == 0
    partner = jnp.where(
        is_lower, pltpu.roll(x, m - d, 0), pltpu.roll(x, d, 0)
    )
    mx = jnp.maximum(x, partner)
    mn = jnp.minimum(x, partner)
    if run is None:
        want_max = (i & d) != 0 if asc else is_lower
    else:
        run_shift = run.bit_length() - 1
        desc = ((i >> run_shift) & 1) == 0
        if flip is not None:
            desc = desc != flip
        want_max = is_lower == desc
    return jnp.where(want_max, mx, mn)


def _sort_chunk(x, chunk, lanes, flip):
    run = 2
    while run <= chunk:
        d = run // 2
        while d >= 1:
            x = _stage(x, d, lanes, run=run, flip=flip)
            d //= 2
        run *= 2
    return x


def _merge_const(m, lanes, asc):
    nc, chunk, _ = m.shape
    x = m.reshape(nc * chunk, lanes)
    d = chunk // 2
    while d >= 1:
        x = _stage(x, d, lanes, asc=asc)
        d //= 2
    return x.reshape(nc, chunk, lanes)


def _merge_tree(x, chunk, nchunks, lanes):
    xr = x.reshape(nchunks // 2, 2, chunk, lanes)
    m = jnp.maximum(xr[:, 0], xr[:, 1])
    nm = nchunks // 2
    while nm > 1:
        mr = m.reshape(nm // 2, 2, chunk, lanes)
        dsc = _merge_const(mr[:, 0], lanes, asc=False)
        acs = _merge_const(mr[:, 1], lanes, asc=True)
        m = jnp.maximum(dsc, acs)
        nm //= 2
    return _merge_const(m, lanes, asc=False).reshape(chunk, lanes)


def _topk_body(len_ref, x_ref, o_ref, scratch, *, s, k, lanes):
    b = pl.program_id(0)
    length = len_ref[b]
    x = x_ref[0]
    row = lax.broadcasted_iota(jnp.int32, (s, 1), 0)
    scratch[:] = jnp.where(row < length, x, _NEG)

    nact = (length + (k - 1)) // k

    def chunk_body(c, carry):
        ch = scratch[pl.ds(c * k, k), :]
        scratch[pl.ds(c * k, k), :] = _sort_chunk(ch, k, lanes, (c & 1) == 1)
        return carry

    lax.fori_loop(0, nact, chunk_body, 0)
    y = _merge_tree(scratch[:], k, s // k, lanes)
    newl = jnp.minimum(length, k)
    orow = lax.broadcasted_iota(jnp.int32, (k, 1), 0)
    o_ref[0] = jnp.where(orow < newl, y, 0.0)


def _build(s, d_total, k, lanes, interpret=False):
    def call(x, lengths):
        bsz = x.shape[0]
        body = functools.partial(_topk_body, s=s, k=k, lanes=lanes)
        return pl.pallas_call(
            body,
            grid=(bsz, d_total // lanes),
            in_specs=[
                pl.BlockSpec(memory_space=pltpu.SMEM),
                pl.BlockSpec((1, s, lanes), lambda b, dt: (b, 0, dt)),
            ],
            out_specs=pl.BlockSpec((1, k, lanes), lambda b, dt: (b, 0, dt)),
            out_shape=jax.ShapeDtypeStruct((bsz, k, d_total), jnp.float32),
            scratch_shapes=[pltpu.VMEM((s, lanes), jnp.float32)],
            compiler_params=pltpu.CompilerParams(
                dimension_semantics=("parallel", "parallel"),
            ),
            interpret=interpret,
        )(lengths, x)

    return call


@jax.jit
def kernel(x, lengths):
    bsz, s, d_total = x.shape
    pooled = _build(s, d_total, _K, 128)(x, lengths)
    return pooled, jnp.minimum(lengths, _K)


# speedup vs baseline: 1.0000x; 1.0000x over previous
import functools

import jax
import jax.numpy as jnp
from jax import lax
from jax.experimental import pallas as pl
from jax.experimental.pallas import tpu as pltpu

_K = 512
_NEG = float("-inf")


def _stage(x, d, lanes, run=None, flip=None, asc=False):
    m = x.shape[0]
    if d >= 8:
        k = m // (2 * d)
        xr = x.reshape(k, 2, d, lanes)
        a = xr[:, 0]
        b = xr[:, 1]
        mx = jnp.maximum(a, b)
        mn = jnp.minimum(a, b)
        if run is None:
            lo, hi = (mn, mx) if asc else (mx, mn)
        else:
            run_shift = run.bit_length() - 1
            shift = run_shift - (2 * d).bit_length() + 1
            blk = lax.broadcasted_iota(jnp.int32, (k, 1, 1), 0)
            desc = ((blk >> shift) & 1) == 0
            if flip is not None:
                desc = desc != flip
            lo = jnp.where(desc, mx, mn)
            hi = jnp.where(desc, mn, mx)
        return jnp.concatenate([lo[:, None], hi[:, None]], axis=1).reshape(
            m, lanes
        )
    i = lax.broadcasted_iota(jnp.int32, (m, lanes), 0)
    is_lower = (i & d) == 0
    partner = jnp.where(
        is_lower, pltpu.roll(x, m - d, 0), pltpu.roll(x, d, 0)
    )
    mx = jnp.maximum(x, partner)
    mn = jnp.minimum(x, partner)
    if run is None:
        want_max = (i & d) != 0 if asc else is_lower
    else:
        run_shift = run.bit_length() - 1
        desc = ((i >> run_shift) & 1) == 0
        if flip is not None:
            desc = desc != flip
        want_max = is_lower == desc
    return jnp.where(want_max, mx, mn)


def _sort_chunk(x, chunk, lanes, flip):
    run = 2
    while run <= chunk:
        d = run // 2
        while d >= 1:
            x = _stage(x, d, lanes, run=run, flip=flip)
            d //= 2
        run *= 2
    return x


def _merge_const(m, lanes, asc):
    nc, chunk, _ = m.shape
    x = m.reshape(nc * chunk, lanes)
    d = chunk // 2
    while d >= 1:
        x = _stage(x, d, lanes, asc=asc)
        d //= 2
    return x.reshape(nc, chunk, lanes)


def _merge_tree(x, chunk, nchunks, lanes):
    xr = x.reshape(nchunks // 2, 2, chunk, lanes)
    m = jnp.maximum(xr[:, 0], xr[:, 1])
    nm = nchunks // 2
    while nm > 1:
        mr = m.reshape(nm // 2, 2, chunk, lanes)
        dsc = _merge_const(mr[:, 0], lanes, asc=False)
        acs = _merge_const(mr[:, 1], lanes, asc=True)
        m = jnp.maximum(dsc, acs)
        nm //= 2
    return _merge_const(m, lanes, asc=False).reshape(chunk, lanes)


def _topk_body(len_ref, x_ref, o_ref, scratch, *, s, k, lanes):
    b = pl.program_id(0)
    length = len_ref[b]
    x = x_ref[0]
    row = lax.broadcasted_iota(jnp.int32, (s, 1), 0)
    scratch[:] = jnp.where(row < length, x, _NEG)

    nact = (length + (k - 1)) // k

    def chunk_body(c, carry):
        ch = scratch[pl.ds(c * k, k), :]
        scratch[pl.ds(c * k, k), :] = _sort_chunk(ch, k, lanes, (c & 1) == 1)
        return carry

    lax.fori_loop(0, nact, chunk_body, 0)
    y = _merge_tree(scratch[:], k, s // k, lanes)
    newl = jnp.minimum(length, k)
    orow = lax.broadcasted_iota(jnp.int32, (k, 1), 0)
    o_ref[0] = jnp.where(orow < newl, y, 0.0)


def _build(s, d_total, k, lanes, interpret=False):
    def call(x, lengths):
        bsz = x.shape[0]
        body = functools.partial(_topk_body, s=s, k=k, lanes=lanes)
        return pl.pallas_call(
            body,
            grid=(bsz, d_total // lanes),
            in_specs=[
                pl.BlockSpec(memory_space=pltpu.SMEM),
                pl.BlockSpec((1, s, lanes), lambda b, dt: (b, 0, dt)),
            ],
            out_specs=pl.BlockSpec((1, k, lanes), lambda b, dt: (b, 0, dt)),
            out_shape=jax.ShapeDtypeStruct((bsz, k, d_total), jnp.float32),
            scratch_shapes=[pltpu.VMEM((s, lanes), jnp.float32)],
            compiler_params=pltpu.CompilerParams(
                dimension_semantics=("parallel", "parallel"),
            ),
            interpret=interpret,
        )(lengths, x)

    return call


@jax.jit
def kernel(x, lengths):
    bsz, s, d_total = x.shape
    pooled = _build(s, d_total, _K, 128)(x, lengths)
    return pooled, jnp.minimum(lengths, _K)


# speedup vs baseline: 1.1412x; 1.1412x over previous
import functools

import jax
import jax.numpy as jnp
from jax import lax
from jax.experimental import pallas as pl
from jax.experimental.pallas import tpu as pltpu

_K = 512
_NEG = float("-inf")


def _stage(x, d, lanes, asc=False):
    m = x.shape[0]
    if d >= 8:
        xr = x.reshape(m // (2 * d), 2, d, lanes)
        a = xr[:, 0]
        b = xr[:, 1]
        mx = jnp.maximum(a, b)
        mn = jnp.minimum(a, b)
        if asc:
            mx, mn = mn, mx
        return jnp.concatenate([mx[:, None], mn[:, None]], axis=1).reshape(
            m, lanes
        )
    i = lax.broadcasted_iota(jnp.int32, (m, lanes), 0)
    is_lower = (i & d) == 0
    y = pltpu.roll(x, m - d, 0)
    hi = jnp.maximum(x, y)
    lo = jnp.minimum(x, y)
    if asc:
        hi, lo = lo, hi
    z = pltpu.roll(lo, d, 0)
    return jnp.where(is_lower, hi, z)


def _sort_chunk(x, chunk, lanes, odd):
    i = lax.broadcasted_iota(jnp.int32, (chunk, lanes), 0)
    neg = ((i >> 1) & 1) == 1
    neg = neg != odd
    x = jnp.where(neg, -x, x)
    run = 2
    while run <= chunk:
        d = run // 2
        while d >= 1:
            x = _stage(x, d, lanes)
            d //= 2
        if run < chunk:
            shift = run.bit_length() - 1
            tb = (((i >> shift) ^ (i >> (shift + 1))) & 1) == 1
            x = jnp.where(tb, -x, x)
        run *= 2
    return jnp.where(odd, -x, x)


def _merge_const(m, lanes, asc):
    nc, chunk, _ = m.shape
    x = m.reshape(nc * chunk, lanes)
    d = chunk // 2
    while d >= 1:
        x = _stage(x, d, lanes, asc=asc)
        d //= 2
    return x.reshape(nc, chunk, lanes)


def _merge_tree(x, chunk, nchunks, lanes):
    xr = x.reshape(nchunks // 2, 2, chunk, lanes)
    m = jnp.maximum(xr[:, 0], xr[:, 1])
    nm = nchunks // 2
    while nm > 1:
        mr = m.reshape(nm // 2, 2, chunk, lanes)
        dsc = _merge_const(mr[:, 0], lanes, asc=False)
        acs = _merge_const(mr[:, 1], lanes, asc=True)
        m = jnp.maximum(dsc, acs)
        nm //= 2
    return _merge_const(m, lanes, asc=False).reshape(chunk, lanes)


def _topk_body(len_ref, x_ref, o_ref, scratch, *, s, k, lanes):
    b = pl.program_id(0)
    length = len_ref[b]
    x = x_ref[0]
    row = lax.broadcasted_iota(jnp.int32, (s, 1), 0)
    scratch[:] = jnp.where(row < length, x, _NEG)

    nact = (length + (k - 1)) // k

    def chunk_body(c, carry):
        ch = scratch[pl.ds(c * k, k), :]
        scratch[pl.ds(c * k, k), :] = _sort_chunk(ch, k, lanes, (c & 1) == 1)
        return carry

    lax.fori_loop(0, nact, chunk_body, 0)
    y = _merge_tree(scratch[:], k, s // k, lanes)
    newl = jnp.minimum(length, k)
    orow = lax.broadcasted_iota(jnp.int32, (k, 1), 0)
    o_ref[0] = jnp.where(orow < newl, y, 0.0)


def _build(s, d_total, k, lanes, interpret=False):
    def call(x, lengths):
        bsz = x.shape[0]
        body = functools.partial(_topk_body, s=s, k=k, lanes=lanes)
        return pl.pallas_call(
            body,
            grid=(bsz, d_total // lanes),
            in_specs=[
                pl.BlockSpec(memory_space=pltpu.SMEM),
                pl.BlockSpec((1, s, lanes), lambda b, dt: (b, 0, dt)),
            ],
            out_specs=pl.BlockSpec((1, k, lanes), lambda b, dt: (b, 0, dt)),
            out_shape=jax.ShapeDtypeStruct((bsz, k, d_total), jnp.float32),
            scratch_shapes=[pltpu.VMEM((s, lanes), jnp.float32)],
            compiler_params=pltpu.CompilerParams(
                dimension_semantics=("parallel", "parallel"),
            ),
            interpret=interpret,
        )(lengths, x)

    return call


@jax.jit
def kernel(x, lengths):
    bsz, s, d_total = x.shape
    pooled = _build(s, d_total, _K, 128)(x, lengths)
    return pooled, jnp.minimum(lengths, _K)


# speedup vs baseline: 1.8522x; 1.6230x over previous
import functools

import jax
import jax.numpy as jnp
from jax import lax
from jax.experimental import pallas as pl
from jax.experimental.pallas import tpu as pltpu

_K = 512
_NEG = float("-inf")


def _pbit(j, nbits):
    return (j + 3) % nbits


def _stage(x, d, lanes, asc=False):
    m = x.shape[0]
    if d >= 8:
        xr = x.reshape(m // (2 * d), 2, d, lanes)
        a = xr[:, 0]
        b = xr[:, 1]
        mx = jnp.maximum(a, b)
        mn = jnp.minimum(a, b)
        if asc:
            mx, mn = mn, mx
        return jnp.concatenate([mx[:, None], mn[:, None]], axis=1).reshape(
            m, lanes
        )
    i = lax.broadcasted_iota(jnp.int32, (m, lanes), 0)
    is_lower = (i & d) == 0
    y = pltpu.roll(x, m - d, 0)
    hi = jnp.maximum(x, y)
    lo = jnp.minimum(x, y)
    if asc:
        hi, lo = lo, hi
    z = pltpu.roll(lo, d, 0)
    return jnp.where(is_lower, hi, z)


def _sort_chunk(x, chunk, lanes, odd):
    nbits = chunk.bit_length() - 1
    i = lax.broadcasted_iota(jnp.int32, (chunk, lanes), 0)
    neg = ((i >> _pbit(1, nbits)) & 1) == 1
    neg = neg != odd
    x = jnp.where(neg, -x, x)
    for r in range(1, nbits + 1):
        for j in range(r - 1, -1, -1):
            x = _stage(x, 1 << _pbit(j, nbits), lanes)
        if r < nbits:
            cur = i >> _pbit(r, nbits)
            nxt = i >> _pbit(r + 1, nbits) if r + 1 < nbits else 0
            tb = ((cur ^ nxt) & 1) == 1
            x = jnp.where(tb, -x, x)
    return jnp.where(odd, -x, x)


def _merge_const(m, lanes, asc):
    nc, chunk, _ = m.shape
    nbits = chunk.bit_length() - 1
    x = m.reshape(nc * chunk, lanes)
    for j in range(nbits - 1, -1, -1):
        x = _stage(x, 1 << _pbit(j, nbits), lanes, asc=asc)
    return x.reshape(nc, chunk, lanes)


def _merge_tree(x, chunk, nchunks, lanes):
    xr = x.reshape(nchunks // 2, 2, chunk, lanes)
    m = jnp.maximum(xr[:, 0], xr[:, 1])
    nm = nchunks // 2
    while nm > 1:
        mr = m.reshape(nm // 2, 2, chunk, lanes)
        dsc = _merge_const(mr[:, 0], lanes, asc=False)
        acs = _merge_const(mr[:, 1], lanes, asc=True)
        m = jnp.maximum(dsc, acs)
        nm //= 2
    return _merge_const(m, lanes, asc=False).reshape(chunk, lanes)


def _unpermute(v, chunk, lanes):
    nbits = chunk.bit_length() - 1
    a = 1 << (nbits - 3) if nbits > 3 else 1
    if a == 1:
        return v
    return jnp.transpose(v.reshape(a, chunk // a, lanes), (1, 0, 2)).reshape(
        chunk, lanes
    )


def _topk_body(len_ref, x_ref, o_ref, scratch, *, s, k, lanes):
    b = pl.program_id(0)
    length = len_ref[b]
    x = x_ref[0]
    row = lax.broadcasted_iota(jnp.int32, (s, 1), 0)
    scratch[:] = jnp.where(row < length, x, _NEG)

    nact = (length + (k - 1)) // k

    def chunk_body(c, carry):
        ch = scratch[pl.ds(c * k, k), :]
        scratch[pl.ds(c * k, k), :] = _sort_chunk(ch, k, lanes, (c & 1) == 1)
        return carry

    lax.fori_loop(0, nact, chunk_body, 0)
    y = _merge_tree(scratch[:], k, s // k, lanes)
    y = _unpermute(y, k, lanes)
    newl = jnp.minimum(length, k)
    orow = lax.broadcasted_iota(jnp.int32, (k, 1), 0)
    o_ref[0] = jnp.where(orow < newl, y, 0.0)


def _build(s, d_total, k, lanes, interpret=False):
    def call(x, lengths):
        bsz = x.shape[0]
        body = functools.partial(_topk_body, s=s, k=k, lanes=lanes)
        return pl.pallas_call(
            body,
            grid=(bsz, d_total // lanes),
            in_specs=[
                pl.BlockSpec(memory_space=pltpu.SMEM),
                pl.BlockSpec((1, s, lanes), lambda b, dt: (b, 0, dt)),
            ],
            out_specs=pl.BlockSpec((1, k, lanes), lambda b, dt: (b, 0, dt)),
            out_shape=jax.ShapeDtypeStruct((bsz, k, d_total), jnp.float32),
            scratch_shapes=[pltpu.VMEM((s, lanes), jnp.float32)],
            compiler_params=pltpu.CompilerParams(
                dimension_semantics=("parallel", "parallel"),
            ),
            interpret=interpret,
        )(lengths, x)

    return call


@jax.jit
def kernel(x, lengths):
    bsz, s, d_total = x.shape
    pooled = _build(s, d_total, _K, 128)(x, lengths)
    return pooled, jnp.minimum(lengths, _K)


# speedup vs baseline: 2.1227x; 1.1461x over previous
import functools

import jax
import jax.numpy as jnp
from jax import lax
from jax.experimental import pallas as pl
from jax.experimental.pallas import tpu as pltpu

_K = 512
_NEG = float("-inf")


def _pbit(j, nbits):
    return (j + 3) % nbits


def _stage(x, d, lanes, asc=False):
    m = x.shape[0]
    if d >= 8:
        xr = x.reshape(m // (2 * d), 2, d, lanes)
        a = xr[:, 0]
        b = xr[:, 1]
        mx = jnp.maximum(a, b)
        mn = jnp.minimum(a, b)
        if asc:
            mx, mn = mn, mx
        return jnp.concatenate([mx[:, None], mn[:, None]], axis=1).reshape(
            m, lanes
        )
    i = lax.broadcasted_iota(jnp.int32, (m, lanes), 0)
    is_lower = (i & d) == 0
    xg = x.reshape(m // 8, 8, lanes)
    y = pltpu.roll(xg, 8 - d, 1).reshape(m, lanes)
    hi = jnp.maximum(x, y)
    lo = jnp.minimum(x, y)
    if asc:
        hi, lo = lo, hi
    z = pltpu.roll(lo.reshape(m // 8, 8, lanes), d, 1).reshape(m, lanes)
    return jnp.where(is_lower, hi, z)


def _sort_chunk(x, chunk, lanes, odd):
    nbits = chunk.bit_length() - 1
    i = lax.broadcasted_iota(jnp.int32, (chunk, lanes), 0)
    neg = ((i >> _pbit(1, nbits)) & 1) == 1
    neg = neg != odd
    x = jnp.where(neg, -x, x)
    for r in range(1, nbits + 1):
        for j in range(r - 1, -1, -1):
            x = _stage(x, 1 << _pbit(j, nbits), lanes)
        if r < nbits:
            cur = i >> _pbit(r, nbits)
            nxt = i >> _pbit(r + 1, nbits) if r + 1 < nbits else 0
            tb = ((cur ^ nxt) & 1) == 1
            x = jnp.where(tb, -x, x)
    return jnp.where(odd, -x, x)


def _merge_const(m, lanes, asc):
    nc, chunk, _ = m.shape
    nbits = chunk.bit_length() - 1
    x = m.reshape(nc * chunk, lanes)
    for j in range(nbits - 1, -1, -1):
        x = _stage(x, 1 << _pbit(j, nbits), lanes, asc=asc)
    return x.reshape(nc, chunk, lanes)


def _merge_tree(x, chunk, nchunks, lanes):
    xr = x.reshape(nchunks // 2, 2, chunk, lanes)
    m = jnp.maximum(xr[:, 0], xr[:, 1])
    nm = nchunks // 2
    while nm > 1:
        mr = m.reshape(nm // 2, 2, chunk, lanes)
        dsc = _merge_const(mr[:, 0], lanes, asc=False)
        acs = _merge_const(mr[:, 1], lanes, asc=True)
        m = jnp.maximum(dsc, acs)
        nm //= 2
    return _merge_const(m, lanes, asc=False).reshape(chunk, lanes)


def _unpermute(v, chunk, lanes):
    nbits = chunk.bit_length() - 1
    a = 1 << (nbits - 3) if nbits > 3 else 1
    if a == 1:
        return v
    return jnp.transpose(v.reshape(a, chunk // a, lanes), (1, 0, 2)).reshape(
        chunk, lanes
    )


def _topk_body(len_ref, x_ref, o_ref, scratch, *, s, k, lanes):
    b = pl.program_id(0)
    length = len_ref[b]
    x = x_ref[0]
    row = lax.broadcasted_iota(jnp.int32, (s, 1), 0)
    scratch[:] = jnp.where(row < length, x, _NEG)

    nact = (length + (k - 1)) // k

    def chunk_body(c, carry):
        ch = scratch[pl.ds(c * k, k), :]
        scratch[pl.ds(c * k, k), :] = _sort_chunk(ch, k, lanes, (c & 1) == 1)
        return carry

    lax.fori_loop(0, nact, chunk_body, 0)
    y = _merge_tree(scratch[:], k, s // k, lanes)
    y = _unpermute(y, k, lanes)
    newl = jnp.minimum(length, k)
    orow = lax.broadcasted_iota(jnp.int32, (k, 1), 0)
    o_ref[0] = jnp.where(orow < newl, y, 0.0)


def _build(s, d_total, k, lanes, interpret=False):
    def call(x, lengths):
        bsz = x.shape[0]
        body = functools.partial(_topk_body, s=s, k=k, lanes=lanes)
        return pl.pallas_call(
            body,
            grid=(bsz, d_total // lanes),
            in_specs=[
                pl.BlockSpec(memory_space=pltpu.SMEM),
                pl.BlockSpec((1, s, lanes), lambda b, dt: (b, 0, dt)),
            ],
            out_specs=pl.BlockSpec((1, k, lanes), lambda b, dt: (b, 0, dt)),
            out_shape=jax.ShapeDtypeStruct((bsz, k, d_total), jnp.float32),
            scratch_shapes=[pltpu.VMEM((s, lanes), jnp.float32)],
            compiler_params=pltpu.CompilerParams(
                dimension_semantics=("parallel", "parallel"),
            ),
            interpret=interpret,
        )(lengths, x)

    return call


@jax.jit
def kernel(x, lengths):
    bsz, s, d_total = x.shape
    pooled = _build(s, d_total, _K, 128)(x, lengths)
    return pooled, jnp.minimum(lengths, _K)


# speedup vs baseline: 2.4757x; 1.1663x over previous
import functools

import jax
import jax.numpy as jnp
from jax import lax
from jax.experimental import pallas as pl
from jax.experimental.pallas import tpu as pltpu

_K = 512
_NEG = float("-inf")


def _pbit(j, nbits):
    return (j + 3) % nbits


def _stage(x, d, lanes, asc=False):
    m = x.shape[0]
    if d >= 8:
        xr = x.reshape(m // (2 * d), 2, d, lanes)
        a = xr[:, 0]
        b = xr[:, 1]
        mx = jnp.maximum(a, b)
        mn = jnp.minimum(a, b)
        if asc:
            mx, mn = mn, mx
        return jnp.concatenate([mx[:, None], mn[:, None]], axis=1).reshape(
            m, lanes
        )
    i = lax.broadcasted_iota(jnp.int32, (m, lanes), 0)
    is_lower = (i & d) == 0
    xg = x.reshape(m // 8, 8, lanes)
    y = pltpu.roll(xg, 8 - d, 1).reshape(m, lanes)
    hi = jnp.maximum(x, y)
    lo = jnp.minimum(x, y)
    if asc:
        hi, lo = lo, hi
    z = pltpu.roll(lo.reshape(m // 8, 8, lanes), d, 1).reshape(m, lanes)
    return jnp.where(is_lower, hi, z)


def _sort_chunk(x, chunk, lanes, odd):
    nbits = chunk.bit_length() - 1
    i = lax.broadcasted_iota(jnp.int32, (chunk, lanes), 0)
    neg = ((i >> _pbit(1, nbits)) & 1) == 1
    neg = neg != odd
    x = jnp.where(neg, -x, x)
    for r in range(1, nbits + 1):
        for j in range(r - 1, -1, -1):
            x = _stage(x, 1 << _pbit(j, nbits), lanes)
        if r < nbits:
            cur = i >> _pbit(r, nbits)
            nxt = i >> _pbit(r + 1, nbits) if r + 1 < nbits else 0
            tb = ((cur ^ nxt) & 1) == 1
            x = jnp.where(tb, -x, x)
    return jnp.where(odd, -x, x)


def _merge_const(m, lanes, asc):
    nc, chunk, _ = m.shape
    nbits = chunk.bit_length() - 1
    x = m.reshape(nc * chunk, lanes)
    for j in range(nbits - 1, -1, -1):
        x = _stage(x, 1 << _pbit(j, nbits), lanes, asc=asc)
    return x.reshape(nc, chunk, lanes)


def _merge_tree(x, chunk, nchunks, lanes):
    xr = x.reshape(nchunks // 2, 2, chunk, lanes)
    m = jnp.maximum(xr[:, 0], xr[:, 1])
    nm = nchunks // 2
    while nm > 1:
        mr = m.reshape(nm // 2, 2, chunk, lanes)
        dsc = _merge_const(mr[:, 0], lanes, asc=False)
        acs = _merge_const(mr[:, 1], lanes, asc=True)
        m = jnp.maximum(dsc, acs)
        nm //= 2
    return _merge_const(m, lanes, asc=False).reshape(chunk, lanes)


def _unpermute(v, chunk, lanes):
    nbits = chunk.bit_length() - 1
    a = 1 << (nbits - 3) if nbits > 3 else 1
    if a == 1:
        return v
    return jnp.transpose(v.reshape(a, chunk // a, lanes), (1, 0, 2)).reshape(
        chunk, lanes
    )


def _topk_body(len_ref, x_ref, o_ref, scratch, *, s, k, lanes):
    b = pl.program_id(0)
    length = len_ref[b]
    x = x_ref[0]
    row = lax.broadcasted_iota(jnp.int32, (s, 1), 0)
    scratch[:] = jnp.where(row < length, x, _NEG)

    nact = (length + (k - 1)) // k

    def chunk_body(c, carry):
        ch = scratch[pl.ds(c * k, k), :]
        scratch[pl.ds(c * k, k), :] = _sort_chunk(ch, k, lanes, (c & 1) == 1)
        return carry

    lax.fori_loop(0, nact, chunk_body, 0)
    y = lax.cond(
        nact <= 2,
        lambda: _merge_tree(scratch[0 : 2 * k, :], k, 2, lanes),
        lambda: lax.cond(
            nact <= 4,
            lambda: _merge_tree(scratch[0 : 4 * k, :], k, 4, lanes),
            lambda: _merge_tree(scratch[:], k, s // k, lanes),
        ),
    )
    y = _unpermute(y, k, lanes)
    newl = jnp.minimum(length, k)
    orow = lax.broadcasted_iota(jnp.int32, (k, 1), 0)
    o_ref[0] = jnp.where(orow < newl, y, 0.0)


def _build(s, d_total, k, lanes, interpret=False):
    def call(x, lengths):
        bsz = x.shape[0]
        body = functools.partial(_topk_body, s=s, k=k, lanes=lanes)
        return pl.pallas_call(
            body,
            grid=(bsz, d_total // lanes),
            in_specs=[
                pl.BlockSpec(memory_space=pltpu.SMEM),
                pl.BlockSpec((1, s, lanes), lambda b, dt: (b, 0, dt)),
            ],
            out_specs=pl.BlockSpec((1, k, lanes), lambda b, dt: (b, 0, dt)),
            out_shape=jax.ShapeDtypeStruct((bsz, k, d_total), jnp.float32),
            scratch_shapes=[pltpu.VMEM((s, lanes), jnp.float32)],
            compiler_params=pltpu.CompilerParams(
                dimension_semantics=("parallel", "parallel"),
            ),
            interpret=interpret,
        )(lengths, x)

    return call


@jax.jit
def kernel(x, lengths):
    bsz, s, d_total = x.shape
    pooled = _build(s, d_total, _K, 128)(x, lengths)
    return pooled, jnp.minimum(lengths, _K)


# speedup vs baseline: 2.7357x; 1.1050x over previous
import functools

import jax
import jax.numpy as jnp
from jax import lax
from jax.experimental import pallas as pl
from jax.experimental.pallas import tpu as pltpu

_K = 512
_NEG = float("-inf")


def _pbit(j, nbits):
    return (j + 3) % nbits


def _stage(x, d, lanes, asc=False):
    m = x.shape[0]
    if d >= 8:
        xr = x.reshape(m // (2 * d), 2, d, lanes)
        a = xr[:, 0]
        b = xr[:, 1]
        mx = jnp.maximum(a, b)
        mn = jnp.minimum(a, b)
        if asc:
            mx, mn = mn, mx
        return jnp.concatenate([mx[:, None], mn[:, None]], axis=1).reshape(
            m, lanes
        )
    i = lax.broadcasted_iota(jnp.int32, (m, lanes), 0)
    is_lower = (i & d) == 0
    xg = x.reshape(m // 8, 8, lanes)
    y = pltpu.roll(xg, 8 - d, 1).reshape(m, lanes)
    hi = jnp.maximum(x, y)
    lo = jnp.minimum(x, y)
    if asc:
        hi, lo = lo, hi
    z = pltpu.roll(lo.reshape(m // 8, 8, lanes), d, 1).reshape(m, lanes)
    return jnp.where(is_lower, hi, z)


def _sort_chunk(x, chunk, lanes, odd):
    nbits = chunk.bit_length() - 1
    i = lax.broadcasted_iota(jnp.int32, (chunk, lanes), 0)
    neg = ((i >> _pbit(1, nbits)) & 1) == 1
    neg = neg != odd
    x = jnp.where(neg, -x, x)
    for r in range(1, nbits + 1):
        for j in range(r - 1, -1, -1):
            x = _stage(x, 1 << _pbit(j, nbits), lanes)
        if r < nbits:
            cur = i >> _pbit(r, nbits)
            nxt = i >> _pbit(r + 1, nbits) if r + 1 < nbits else 0
            tb = ((cur ^ nxt) & 1) == 1
            x = jnp.where(tb, -x, x)
    return jnp.where(odd, -x, x)


def _merge_const(m, lanes, asc):
    nc, chunk, _ = m.shape
    nbits = chunk.bit_length() - 1
    x = m.reshape(nc * chunk, lanes)
    for j in range(nbits - 1, -1, -1):
        x = _stage(x, 1 << _pbit(j, nbits), lanes, asc=asc)
    return x.reshape(nc, chunk, lanes)


def _merge_tree(x, chunk, nchunks, lanes):
    xr = x.reshape(nchunks // 2, 2, chunk, lanes)
    m = jnp.maximum(xr[:, 0], xr[:, 1])
    nm = nchunks // 2
    while nm > 1:
        mr = m.reshape(nm // 2, 2, chunk, lanes)
        dsc = _merge_const(mr[:, 0], lanes, asc=False)
        acs = _merge_const(mr[:, 1], lanes, asc=True)
        m = jnp.maximum(dsc, acs)
        nm //= 2
    return _merge_const(m, lanes, asc=False).reshape(chunk, lanes)


def _unpermute(v, chunk, lanes):
    nbits = chunk.bit_length() - 1
    a = 1 << (nbits - 3) if nbits > 3 else 1
    if a == 1:
        return v
    return jnp.transpose(v.reshape(a, chunk // a, lanes), (1, 0, 2)).reshape(
        chunk, lanes
    )


def _topk_body(len_ref, x_ref, o_ref, scratch, *, s, k, lanes):
    b = pl.program_id(0)
    length = len_ref[b]
    x = x_ref[0]
    row = lax.broadcasted_iota(jnp.int32, (s, 1), 0)
    scratch[:] = jnp.where(row < length, x, _NEG)

    nact = (length + (k - 1)) // k

    def chunk_body(c, carry):
        ch = scratch[pl.ds(c * k, k), :]
        scratch[pl.ds(c * k, k), :] = _sort_chunk(ch, k, lanes, (c & 1) == 1)
        return carry

    lax.fori_loop(0, nact, chunk_body, 0)
    y = lax.cond(
        nact <= 2,
        lambda: _merge_tree(scratch[0 : 2 * k, :], k, 2, lanes),
        lambda: lax.cond(
            nact <= 4,
            lambda: _merge_tree(scratch[0 : 4 * k, :], k, 4, lanes),
            lambda: _merge_tree(scratch[:], k, s // k, lanes),
        ),
    )
    y = _unpermute(y, k, lanes)
    newl = jnp.minimum(length, k)
    orow = lax.broadcasted_iota(jnp.int32, (k, 1), 0)
    o_ref[0] = jnp.where(orow < newl, y, 0.0)


def _build(s, d_total, k, lanes, interpret=False):
    def call(x, lengths):
        bsz = x.shape[0]
        body = functools.partial(_topk_body, s=s, k=k, lanes=lanes)
        return pl.pallas_call(
            body,
            grid=(bsz, d_total // lanes),
            in_specs=[
                pl.BlockSpec(memory_space=pltpu.SMEM),
                pl.BlockSpec((1, s, lanes), lambda b, dt: (b, 0, dt)),
            ],
            out_specs=pl.BlockSpec((1, k, lanes), lambda b, dt: (b, 0, dt)),
            out_shape=jax.ShapeDtypeStruct((bsz, k, d_total), jnp.float32),
            scratch_shapes=[pltpu.VMEM((s, lanes), jnp.float32)],
            compiler_params=pltpu.CompilerParams(
                dimension_semantics=("parallel", "parallel"),
            ),
            interpret=interpret,
        )(lengths, x)

    return call


@jax.jit
def kernel(x, lengths):
    bsz, s, d_total = x.shape
    pooled = _build(s, d_total, _K, 256)(x, lengths)
    return pooled, jnp.minimum(lengths, _K)


# speedup vs baseline: 2.8734x; 1.0503x over previous
import functools

import jax
import jax.numpy as jnp
from jax import lax
from jax.experimental import pallas as pl
from jax.experimental.pallas import tpu as pltpu

_K = 512
_NEG = float("-inf")


def _pbit(j, nbits):
    return (j + 3) % nbits


def _stage(x, d, lanes, asc=False):
    m = x.shape[0]
    if d >= 8:
        xr = x.reshape(m // (2 * d), 2, d, lanes)
        a = xr[:, 0]
        b = xr[:, 1]
        mx = jnp.maximum(a, b)
        mn = jnp.minimum(a, b)
        if asc:
            mx, mn = mn, mx
        return jnp.concatenate([mx[:, None], mn[:, None]], axis=1).reshape(
            m, lanes
        )
    i = lax.broadcasted_iota(jnp.int32, (m, lanes), 0)
    is_lower = (i & d) == 0
    xg = x.reshape(m // 8, 8, lanes)
    y = pltpu.roll(xg, 8 - d, 1).reshape(m, lanes)
    hi = jnp.maximum(x, y)
    lo = jnp.minimum(x, y)
    if asc:
        hi, lo = lo, hi
    z = pltpu.roll(lo.reshape(m // 8, 8, lanes), d, 1).reshape(m, lanes)
    return jnp.where(is_lower, hi, z)


def _sort_chunk(x, chunk, lanes, odd):
    nbits = chunk.bit_length() - 1
    i = lax.broadcasted_iota(jnp.int32, (chunk, lanes), 0)
    neg = ((i >> _pbit(1, nbits)) & 1) == 1
    neg = neg != odd
    x = jnp.where(neg, -x, x)
    for r in range(1, nbits + 1):
        for j in range(r - 1, -1, -1):
            x = _stage(x, 1 << _pbit(j, nbits), lanes)
        if r < nbits:
            cur = i >> _pbit(r, nbits)
            nxt = i >> _pbit(r + 1, nbits) if r + 1 < nbits else 0
            tb = ((cur ^ nxt) & 1) == 1
            x = jnp.where(tb, -x, x)
    return jnp.where(odd, -x, x)


def _merge_const(m, lanes, asc):
    nc, chunk, _ = m.shape
    nbits = chunk.bit_length() - 1
    x = m.reshape(nc * chunk, lanes)
    for j in range(nbits - 1, -1, -1):
        x = _stage(x, 1 << _pbit(j, nbits), lanes, asc=asc)
    return x.reshape(nc, chunk, lanes)


def _merge_tree(x, chunk, nchunks, lanes):
    xr = x.reshape(nchunks // 2, 2, chunk, lanes)
    m = jnp.maximum(xr[:, 0], xr[:, 1])
    nm = nchunks // 2
    while nm > 1:
        mr = m.reshape(nm // 2, 2, chunk, lanes)
        dsc = _merge_const(mr[:, 0], lanes, asc=False)
        acs = _merge_const(mr[:, 1], lanes, asc=True)
        m = jnp.maximum(dsc, acs)
        nm //= 2
    return _merge_const(m, lanes, asc=False).reshape(chunk, lanes)


def _unpermute(v, chunk, lanes):
    nbits = chunk.bit_length() - 1
    a = 1 << (nbits - 3) if nbits > 3 else 1
    if a == 1:
        return v
    return jnp.transpose(v.reshape(a, chunk // a, lanes), (1, 0, 2)).reshape(
        chunk, lanes
    )


def _topk_body(len_ref, x_ref, o_ref, scratch, *, s, k, lanes):
    b = pl.program_id(0)
    length = len_ref[b]
    x = x_ref[0]
    row = lax.broadcasted_iota(jnp.int32, (s, 1), 0)
    scratch[:] = jnp.where(row < length, x, _NEG)

    nact = (length + (k - 1)) // k

    def chunk_body(c, carry):
        ch = scratch[pl.ds(c * k, k), :]
        scratch[pl.ds(c * k, k), :] = _sort_chunk(ch, k, lanes, (c & 1) == 1)
        return carry

    lax.fori_loop(0, nact, chunk_body, 0)
    y = lax.cond(
        nact <= 2,
        lambda: _merge_tree(scratch[0 : 2 * k, :], k, 2, lanes),
        lambda: lax.cond(
            nact <= 4,
            lambda: _merge_tree(scratch[0 : 4 * k, :], k, 4, lanes),
            lambda: _merge_tree(scratch[:], k, s // k, lanes),
        ),
    )
    y = _unpermute(y, k, lanes)
    newl = jnp.minimum(length, k)
    orow = lax.broadcasted_iota(jnp.int32, (k, 1), 0)
    o_ref[0] = jnp.where(orow < newl, y, 0.0)


def _build(s, d_total, k, lanes, interpret=False):
    def call(x, lengths):
        bsz = x.shape[0]
        body = functools.partial(_topk_body, s=s, k=k, lanes=lanes)
        return pl.pallas_call(
            body,
            grid=(bsz, d_total // lanes),
            in_specs=[
                pl.BlockSpec(memory_space=pltpu.SMEM),
                pl.BlockSpec((1, s, lanes), lambda b, dt: (b, 0, dt)),
            ],
            out_specs=pl.BlockSpec((1, k, lanes), lambda b, dt: (b, 0, dt)),
            out_shape=jax.ShapeDtypeStruct((bsz, k, d_total), jnp.float32),
            scratch_shapes=[pltpu.VMEM((s, lanes), jnp.float32)],
            compiler_params=pltpu.CompilerParams(
                dimension_semantics=("parallel", "parallel"),
            ),
            interpret=interpret,
        )(lengths, x)

    return call


@jax.jit
def kernel(x, lengths):
    bsz, s, d_total = x.shape
    pooled = _build(s, d_total, _K, 512)(x, lengths)
    return pooled, jnp.minimum(lengths, _K)
